# Initial kernel scaffold; baseline (speedup 1.0000x reference)
import jax
import jax.numpy as jnp
from jax.experimental import pallas as pl


def kernel(x, v, grad_row, grad_col, grad_val, div_row, div_col, div_val, edge_index, W_smax, b_smax, g_smax, be_smax, W_s, b_s, g_s, be_s, W_v, g_v, be_v):
    raise NotImplementedError("write your pallas kernel here")


# speedup vs baseline: 1.0150x; 1.0150x over previous
import functools

import jax
import jax.numpy as jnp
from jax.experimental import pallas as pl
from jax.experimental.pallas import tpu as pltpu


def _mm_body(x_ref, w_ref, b_ref, o_ref, *, relu):
    acc = jnp.dot(x_ref[...], w_ref[...], preferred_element_type=jnp.float32)
    acc = acc + b_ref[...]
    if relu:
        acc = jnp.maximum(acc, 0.0)
    o_ref[...] = acc


def _mm(x, w, b, relu, block_rows):
    r, k = x.shape
    c = w.shape[1]
    grid = (r // block_rows,)
    return pl.pallas_call(
        functools.partial(_mm_body, relu=relu),
        grid=grid,
        in_specs=[
            pl.BlockSpec((block_rows, k), lambda i: (i, 0)),
            pl.BlockSpec((k, c), lambda i: (0, 0)),
            pl.BlockSpec((1, c), lambda i: (0, 0)),
        ],
        out_specs=pl.BlockSpec((block_rows, c), lambda i: (i, 0)),
        out_shape=jax.ShapeDtypeStruct((r, c), jnp.float32),
    )(x, w, b)


def _J(v):
    vv = v.reshape(-1, 2, v.shape[-1])
    return jnp.stack([-vv[:, 1], vv[:, 0]], axis=1).reshape(v.shape)


def _norm(v):
    vv = v.reshape(-1, 2, v.shape[-1])
    return jnp.sqrt(jnp.sum(vv * vv, axis=1) + 1e-12)


def _spmm(rows, cols, vals, B, n_rows):
    return jax.ops.segment_sum(vals[:, None] * jnp.take(B, cols, axis=0), rows,
                               num_segments=n_rows)


def kernel(x, v, grad_row, grad_col, grad_val, div_row, div_col, div_val,
           edge_index, W_smax, b_smax, g_smax, be_smax, W_s, b_s, g_s, be_s,
           W_v, g_v, be_v):
    n = x.shape[0]
    A1 = (g_smax[:, None] * W_smax).T
    c1 = (g_smax * b_smax + be_smax)[None, :]
    A2 = (g_s[:, None] * W_s).T
    c2 = (g_s * b_s + be_s)[None, :]
    A3 = W_v.T
    c3 = jnp.zeros((1, W_v.shape[0]), jnp.float32)

    smax = _mm(x, A1, c1, relu=True, block_rows=400)
    x_max = jax.ops.segment_max(jnp.take(smax, edge_index[1], axis=0),
                                edge_index[0], num_segments=n)
    x_max = jnp.where(jnp.isneginf(x_max), 0.0, x_max)

    div_v = _spmm(div_row, div_col, div_val, v, n)
    curl_v = _spmm(div_row, div_col, div_val, _J(v), n)
    x_cat = jnp.concatenate([x, div_v, curl_v, _norm(v)], axis=1)
    x_out = x_max + _mm(x_cat, A2, c2, relu=True, block_rows=400)

    hodge = (_spmm(grad_row, grad_col, grad_val, div_v, 2 * n)
             + _J(_spmm(grad_row, grad_col, grad_val, curl_v, 2 * n)))
    grad_x = _spmm(grad_row, grad_col, grad_val, x_out, 2 * n)
    v_cat = jnp.concatenate([v, hodge, grad_x], axis=1)
    v6 = jnp.concatenate([v_cat, _J(v_cat)], axis=1)
    h = _mm(v6, A3, c3, relu=False, block_rows=400)
    nrm = _norm(h)
    scale = jax.nn.relu(g_v * nrm + be_v) / (nrm + 1e-12)
    v_out = h * jnp.repeat(scale, 2, axis=0)
    return (x_out, v_out)


# speedup vs baseline: 2.8307x; 2.7888x over previous
import functools

import jax
import jax.numpy as jnp
from jax import lax
from jax.experimental import pallas as pl
from jax.experimental.pallas import tpu as pltpu
from jax.experimental.pallas import tpu_sc as plsc

N = 10000
NNZ = 640000
E = 320000
SCAN_B = 4000
GB = 64
CH = 128


def _sslice(vec, j):
    return lax.squeeze(lax.slice(vec, (j,), (j + 1,)), (0,))


def _seg_body(n_out, d, nspans, is_max, nnz, b2_hbm, args):
    if is_max:
        (rows_hbm, cols_hbm, out_hbm, stage0, stage1, lpack, cidx0, cidx1,
         widx, scr_r, scr_c, sem0, sem1, semr, semc, acc) = args
        vals_hbm = lval = scr_v = semv = None
    else:
        (rows_hbm, cols_hbm, vals_hbm, out_hbm, stage0, stage1, lpack, lval,
         cidx0, cidx1, widx, scr_r, scr_c, scr_v, sem0, sem1, semr, semc,
         semv, acc) = args
    chs = d // CH
    span = (n_out // nspans) & ~7
    tailr = n_out - (nspans - 1) * span
    c = lax.axis_index("c")
    s = lax.axis_index("s")
    wid = c * 16 + s
    zf = jnp.zeros((16,), jnp.float32)
    zi = jnp.zeros((16,), jnp.int32)
    for p in range(nspans // 32):
        vwid = wid + p * 32
        wlo = vwid * span
        myspan = jnp.where(vwid == nspans - 1, tailr, span)
        _seg_span(n_out, d, chs, span, tailr, is_max, nnz, b2_hbm, wlo,
                  myspan, rows_hbm, cols_hbm, vals_hbm, out_hbm, stage0,
                  stage1, lpack, lval, cidx0, cidx1, widx, scr_r, scr_c,
                  scr_v, sem0, sem1, semr, semc, semv, acc, zf, zi)


def _seg_span(n_out, d, chs, span, tailr, is_max, nnz, b2_hbm, wlo, myspan,
              rows_hbm, cols_hbm, vals_hbm, out_hbm, stage0, stage1, lpack,
              lval, cidx0, cidx1, widx, scr_r, scr_c, scr_v, sem0, sem1,
              semr, semc, semv, acc, zf, zi):
    accr = -(-(tailr + 8) // GB) * GB

    def _scan_step(k, m):
        o = k * SCAN_B
        cr = pltpu.async_copy(rows_hbm.at[pl.ds(o, SCAN_B)], scr_r, semr)
        cc_ = pltpu.async_copy(cols_hbm.at[pl.ds(o, SCAN_B)], scr_c, semc)
        if not is_max:
            cv = pltpu.async_copy(vals_hbm.at[pl.ds(o, SCAN_B)], scr_v, semv)
        cr.wait()
        cc_.wait()
        if not is_max:
            cv.wait()

        def _scan_vec(i, m):
            r = scr_r[pl.ds(i * 16, 16)]
            cc = scr_c[pl.ds(i * 16, 16)]
            rl = r - wlo
            msk = (rl >= 0) & (rl < myspan)
            pos = m + plsc.cumsum(jnp.where(msk, 1, 0)) - 1
            plsc.store_scatter(lpack, [pos], rl * 32768 + cc, mask=msk)
            if not is_max:
                vv = scr_v[pl.ds(i * 16, 16)]
                plsc.store_scatter(lval, [pos], vv, mask=msk)
            cnt = plsc.all_reduce_population_count(msk)
            return m + _sslice(cnt, 0)
        return lax.fori_loop(0, SCAN_B // 16, _scan_vec, m)
    m = lax.fori_loop(0, nnz // SCAN_B, _scan_step, jnp.int32(0))

    dump16 = zi + tailr * 32768
    for k in range(2 * GB // 16):
        lpack[pl.ds(m + k * 16, 16)] = dump16
        if not is_max:
            lval[pl.ds(m + k * 16, 16)] = zf
    npairs = (m + 2 * GB - 1) // (2 * GB)

    ii16 = lax.iota(jnp.int32, 16)

    for h in range(chs):
        def _build(cbuf, j, h=h):
            for k in range(GB // 16):
                p = lpack[pl.ds(j * GB + k * 16, 16)]
                cbuf[pl.ds(k * 16, 16)] = (p & 32767) * chs + h

        def _zacc(i, _):
            for k in range(CH // 16):
                acc[i, pl.ds(k * 16, 16)] = zf
            return 0
        lax.fori_loop(0, accr, _zacc, 0)

        def _rmw(buf, j):
            def _grp(g, _):
                p16 = lpack[pl.ds(j * GB + g * 16, 16)]
                rl16 = lax.shift_right_logical(p16, 15)
                if not is_max:
                    vv16 = lval[pl.ds(j * GB + g * 16, 16)]
                for jj in range(16):
                    rl = _sslice(rl16, jj)
                    row = g * 16 + jj
                    if not is_max:
                        vs = _sslice(vv16, jj)
                    for k in range(CH // 16):
                        a = acc[rl, pl.ds(k * 16, 16)]
                        v = buf[row, pl.ds(k * 16, 16)]
                        if is_max:
                            acc[rl, pl.ds(k * 16, 16)] = jnp.maximum(a, v)
                        else:
                            acc[rl, pl.ds(k * 16, 16)] = a + v * vs
                return 0
            lax.fori_loop(0, GB // 16, _grp, 0)

        _build(cidx0, jnp.int32(0))
        pltpu.async_copy(b2_hbm.at[cidx0], stage0, sem0)

        def _pair(k2, _):
            pltpu.make_async_copy(b2_hbm.at[cidx0], stage0, sem0).wait()
            _build(cidx1, 2 * k2 + 1)
            pltpu.async_copy(b2_hbm.at[cidx1], stage1, sem1)
            _rmw(stage0, 2 * k2)
            pltpu.make_async_copy(b2_hbm.at[cidx1], stage1, sem1).wait()

            @pl.when(k2 + 1 < npairs)
            def _():
                _build(cidx0, 2 * k2 + 2)
                pltpu.async_copy(b2_hbm.at[cidx0], stage0, sem0)
            _rmw(stage1, 2 * k2 + 1)
            return 0
        lax.fori_loop(0, npairs, _pair, 0)

        def _wout(w, _):
            for g in range(GB // 16):
                loc = w * GB + g * 16 + ii16
                gr = (wlo + loc) * chs + h
                widx[pl.ds(g * 16, 16)] = jnp.where(loc < myspan, gr,
                                                    n_out * chs)
            pltpu.sync_copy(acc.at[pl.ds(w * GB, GB)], out_hbm.at[widx])
            return 0
        lax.fori_loop(0, (myspan + GB - 1) // GB, _wout, 0)


def _sc_seg(b, rows, cols, vals, n_out, d, cap, nspans, is_max):
    chs = d // CH
    span = (n_out // nspans) & ~7
    tailr = n_out - (nspans - 1) * span
    accr = -(-(tailr + 8) // GB) * GB
    nnz = rows.shape[0]
    mesh = plsc.VectorSubcoreMesh(core_axis_name="c", subcore_axis_name="s")

    def body(b2, *args):
        _seg_body(n_out, d, nspans, is_max, nnz, b2, args)

    scratch = [
        pltpu.VMEM((GB, CH), jnp.float32),
        pltpu.VMEM((GB, CH), jnp.float32),
        pltpu.VMEM((cap + 2 * GB,), jnp.int32),
    ]
    if not is_max:
        scratch.append(pltpu.VMEM((cap + 2 * GB,), jnp.float32))
    scratch += [
        pltpu.VMEM((GB,), jnp.int32),
        pltpu.VMEM((GB,), jnp.int32),
        pltpu.VMEM((GB,), jnp.int32),
        pltpu.VMEM((SCAN_B,), jnp.int32),
        pltpu.VMEM((SCAN_B,), jnp.int32),
    ]
    if not is_max:
        scratch.append(pltpu.VMEM((SCAN_B,), jnp.float32))
    scratch += [pltpu.SemaphoreType.DMA, pltpu.SemaphoreType.DMA,
                pltpu.SemaphoreType.DMA, pltpu.SemaphoreType.DMA]
    if not is_max:
        scratch.append(pltpu.SemaphoreType.DMA)
    scratch.append(pltpu.VMEM((accr, CH), jnp.float32))

    f = pl.kernel(
        body,
        out_type=jax.ShapeDtypeStruct((n_out * chs + 16, CH), jnp.float32),
        mesh=mesh,
        compiler_params=pltpu.CompilerParams(needs_layout_passes=False),
        scratch_types=scratch,
    )
    if is_max:
        return f(b, rows, cols)
    return f(b, rows, cols, vals)


def _seg_sum(b, rows, cols, vals, n_out, d, cap, nspans):
    chs = d // CH
    b2 = b.reshape(-1, CH)
    o = _sc_seg(b2, rows, cols, vals, n_out, d, cap, nspans, is_max=False)
    return o[:n_out * chs].reshape(n_out, d)


def _seg_max(b, rows, cols, n_out, d, cap, nspans):
    chs = d // CH
    b2 = b.reshape(-1, CH)
    o = _sc_seg(b2, rows, cols, None, n_out, d, cap, nspans, is_max=True)
    return o[:n_out * chs].reshape(n_out, d)


def _mm_body(x_ref, w_ref, b_ref, o_ref, *, relu):
    acc = jnp.dot(x_ref[...], w_ref[...], preferred_element_type=jnp.float32)
    acc = acc + b_ref[...]
    if relu:
        acc = jnp.maximum(acc, 0.0)
    o_ref[...] = acc


def _mm(x, w, b, relu, block_rows=400):
    r, k = x.shape
    c = w.shape[1]
    return pl.pallas_call(
        functools.partial(_mm_body, relu=relu),
        grid=(r // block_rows,),
        in_specs=[
            pl.BlockSpec((block_rows, k), lambda i: (i, 0)),
            pl.BlockSpec((k, c), lambda i: (0, 0)),
            pl.BlockSpec((1, c), lambda i: (0, 0)),
        ],
        out_specs=pl.BlockSpec((block_rows, c), lambda i: (i, 0)),
        out_shape=jax.ShapeDtypeStruct((r, c), jnp.float32),
    )(x, w, b)


def _J(v):
    vv = v.reshape(-1, 2, v.shape[-1])
    return jnp.stack([-vv[:, 1], vv[:, 0]], axis=1).reshape(v.shape)


def _norm(v):
    vv = v.reshape(-1, 2, v.shape[-1])
    return jnp.sqrt(jnp.sum(vv * vv, axis=1) + 1e-12)


def kernel(x, v, grad_row, grad_col, grad_val, div_row, div_col, div_val,
           edge_index, W_smax, b_smax, g_smax, be_smax, W_s, b_s, g_s, be_s,
           W_v, g_v, be_v):
    n = x.shape[0]
    A1 = (g_smax[:, None] * W_smax).T
    c1 = (g_smax * b_smax + be_smax)[None, :]
    A2 = (g_s[:, None] * W_s).T
    c2 = (g_s * b_s + be_s)[None, :]

    smax = _mm(x, A1, c1, relu=True)
    x_max = _seg_max(smax, edge_index[0].astype(jnp.int32),
                     edge_index[1].astype(jnp.int32), n, 128, cap=12288,
                     nspans=32)

    B1 = jnp.concatenate([v, _J(v)], axis=1)
    DV = _seg_sum(B1, div_row.astype(jnp.int32), div_col.astype(jnp.int32),
                  div_val, n, 256, cap=23040, nspans=32)
    div_v, curl_v = DV[:, :128], DV[:, 128:]

    x_cat = jnp.concatenate([x, div_v, curl_v, _norm(v)], axis=1)
    x_out = x_max + _mm(x_cat, A2, c2, relu=True)

    C = x.shape[1]
    W1, W2, W3, W4, W5, W6 = (W_v[:, i * C:(i + 1) * C] for i in range(6))
    P = div_v @ W2.T - curl_v @ W5.T + x_out @ W3.T
    Q = curl_v @ W2.T + div_v @ W5.T + x_out @ W6.T
    R = jnp.concatenate([P, Q], axis=1)
    U = _seg_sum(R, grad_row.astype(jnp.int32), grad_col.astype(jnp.int32),
                 grad_val, 2 * n, 256, cap=12800, nspans=64)
    h = v @ W1.T + _J(v @ W4.T) + U[:, :128] + _J(U[:, 128:])
    nrm = _norm(h)
    scale = jax.nn.relu(g_v * nrm + be_v) / (nrm + 1e-12)
    v_out = h * jnp.repeat(scale, 2, axis=0)
    return (x_out, v_out)
